# baseline (speedup 1.0000x reference)
import jax
import jax.numpy as jnp
from jax import lax
from jax.experimental import pallas as pl
from jax.experimental.pallas import tpu as pltpu
from jax.experimental.pallas import tpu_sc as plsc

B, S, D = 4, 4096, 2048
NC, NS = 2, 16
NW = NC * NS
DW = 256
ND = D // DW
NV = DW // 16
S_CHUNK = 64
N_CHUNK = S // S_CHUNK
NB = 3
N_GROUP = N_CHUNK // NB


def _scan_chunk(in_ref, out_ref, carry):
    def row(s, carry):
        new = []
        for j in range(NV):
            x = in_ref[s, pl.ds(j * 16, 16)]
            acc = carry[j] + x
            out_ref[s, pl.ds(j * 16, 16)] = acc
            new.append(acc)
        return tuple(new)

    return lax.fori_loop(0, S_CHUNK, row, carry)


def _cumsum_body(val_hbm, out_hbm, ins, outs, sls, sss):
    wid = lax.axis_index("s") * NC + lax.axis_index("c")
    b = wid // ND
    d0 = pl.multiple_of((wid % ND) * DW, DW)

    def hbm_in(c):
        s0 = pl.multiple_of(c * S_CHUNK, S_CHUNK)
        return val_hbm.at[b, pl.ds(s0, S_CHUNK), pl.ds(d0, DW)]

    def hbm_out(c):
        s0 = pl.multiple_of(c * S_CHUNK, S_CHUNK)
        return out_hbm.at[b, pl.ds(s0, S_CHUNK), pl.ds(d0, DW)]

    def start_load(c, k):
        pltpu.async_copy(hbm_in(c), ins[k], sls[k])

    def wait_load(c, k):
        pltpu.make_async_copy(hbm_in(c), ins[k], sls[k]).wait()

    def start_store(c, k):
        pltpu.async_copy(outs[k], hbm_out(c), sss[k])

    def wait_store(c, k):
        pltpu.make_async_copy(outs[k], hbm_out(c), sss[k]).wait()

    for c in range(NB - 1):
        start_load(c, c % NB)
    carry0 = tuple(jnp.zeros((16,), jnp.float32) for _ in range(NV))

    def group(i, carry):
        for k in range(NB):
            c = i * NB + k

            @pl.when(c + NB - 1 < N_CHUNK)
            def _():
                start_load(c + NB - 1, (k + NB - 1) % NB)

            wait_load(c, k)

            @pl.when(c >= NB)
            def _():
                wait_store(c - NB, k)

            carry = _scan_chunk(ins[k], outs[k], carry)
            start_store(c, k)
        return carry

    carry = lax.fori_loop(0, N_GROUP, group, carry0)

    kt = (N_CHUNK - 1) % NB
    wait_load(N_CHUNK - 1, kt)
    wait_store(N_CHUNK - 1 - NB, kt)
    carry = _scan_chunk(ins[kt], outs[kt], carry)
    start_store(N_CHUNK - 1, kt)

    for c in range(N_CHUNK - NB, N_CHUNK):
        wait_store(c, c % NB)


@jax.jit
def kernel(val):
    mesh = plsc.VectorSubcoreMesh(core_axis_name="c", subcore_axis_name="s")

    def body(val_hbm, out_hbm, *scratch):
        _cumsum_body(val_hbm, out_hbm, scratch[:NB], scratch[NB:2 * NB],
                     scratch[2 * NB:3 * NB], scratch[3 * NB:])

    f = pl.kernel(
        body,
        out_type=jax.ShapeDtypeStruct((B, S, D), jnp.float32),
        mesh=mesh,
        scratch_types=(
            [pltpu.VMEM((S_CHUNK, DW), jnp.float32)] * (2 * NB)
            + [pltpu.SemaphoreType.DMA] * (2 * NB)
        ),
    )
    return f(val)
